# baseline (speedup 1.0000x reference)
import functools

import jax
import jax.numpy as jnp
from jax.experimental import pallas as pl

_CELL_BLOCK = 128
_GENE_CHUNK = 128


def _tc_body(x_ref, wf_ref, bg_ref, o_ref, *, genes: int, dim: int):
    chunk_k = _GENE_CHUNK * dim
    k_iota = jax.lax.broadcasted_iota(jnp.int32, (chunk_k, _GENE_CHUNK), 0)
    g_iota = jax.lax.broadcasted_iota(jnp.int32, (chunk_k, _GENE_CHUNK), 1)
    sel = (k_iota // dim == g_iota).astype(jnp.bfloat16)
    dot_dims = (((1,), (0,)), ((), ()))

    n_full = genes // _GENE_CHUNK
    for i in range(n_full):
        k0 = i * chunk_k
        y = x_ref[:, k0:k0 + chunk_k] * wf_ref[:, k0:k0 + chunk_k]
        acc = jax.lax.dot_general(
            y.astype(jnp.bfloat16), sel, dot_dims,
            preferred_element_type=jnp.float32)
        g0 = i * _GENE_CHUNK
        o_ref[:, g0:g0 + _GENE_CHUNK] = acc + bg_ref[:, g0:g0 + _GENE_CHUNK]

    rem_g = genes - n_full * _GENE_CHUNK
    if rem_g:
        rem_k = rem_g * dim
        k0 = n_full * chunk_k
        g0 = n_full * _GENE_CHUNK
        y = x_ref[:, k0:k0 + rem_k] * wf_ref[:, k0:k0 + rem_k]
        acc = jax.lax.dot_general(
            y.astype(jnp.bfloat16), sel[:rem_k, :], dot_dims,
            preferred_element_type=jnp.float32)
        o_ref[:, g0:g0 + rem_g] = acc[:, :rem_g] + bg_ref[:, g0:g0 + rem_g]


def _expression_tc(x2, wf, bg, *, interpret=False):
    cells, gd = x2.shape
    genes = bg.shape[1]
    dim = gd // genes
    body = functools.partial(_tc_body, genes=genes, dim=dim)
    return pl.pallas_call(
        body,
        grid=(cells // _CELL_BLOCK,),
        in_specs=[
            pl.BlockSpec((_CELL_BLOCK, gd), lambda i: (i, 0)),
            pl.BlockSpec((1, gd), lambda i: (0, 0)),
            pl.BlockSpec((1, genes), lambda i: (0, 0)),
        ],
        out_specs=pl.BlockSpec((_CELL_BLOCK, genes), lambda i: (i, 0)),
        out_shape=jax.ShapeDtypeStruct((cells, genes), jnp.float32),
        interpret=interpret,
    )(x2, wf, bg)


def kernel(cell_gene_embedding, gene_ix, weight1, bias1):
    cells, genes, dim = cell_gene_embedding.shape
    wf = jnp.take(weight1, gene_ix, axis=0).reshape(1, genes * dim)
    bg = jnp.take(bias1, gene_ix, axis=0).reshape(1, genes)
    x2 = cell_gene_embedding.reshape(cells, genes * dim)
    return _expression_tc(x2, wf, bg)


# speedup vs baseline: 6.4014x; 6.4014x over previous
import functools

import jax
import jax.numpy as jnp
from jax.experimental import pallas as pl

_GENE_BLOCK = 200


def _tc_body(xt_ref, wt_ref, b_ref, o_ref, *, dim: int):
    acc = b_ref[...]
    acc = acc + xt_ref[0] * wt_ref[0]
    for d in range(1, dim):
        acc = acc + xt_ref[d] * wt_ref[d]
    o_ref[...] = acc


def _expression_tc(xt, wt, bg):
    dim, genes, cells = xt.shape
    body = functools.partial(_tc_body, dim=dim)
    out_t = pl.pallas_call(
        body,
        grid=(genes // _GENE_BLOCK,),
        in_specs=[
            pl.BlockSpec((dim, _GENE_BLOCK, cells), lambda j: (0, j, 0)),
            pl.BlockSpec((dim, _GENE_BLOCK, 1), lambda j: (0, j, 0)),
            pl.BlockSpec((_GENE_BLOCK, 1), lambda j: (j, 0)),
        ],
        out_specs=pl.BlockSpec((_GENE_BLOCK, cells), lambda j: (j, 0)),
        out_shape=jax.ShapeDtypeStruct((genes, cells), jnp.float32),
    )(xt, wt, bg)
    return out_t


def kernel(cell_gene_embedding, gene_ix, weight1, bias1):
    cells, genes, dim = cell_gene_embedding.shape
    xt = jnp.transpose(cell_gene_embedding, (2, 1, 0))
    wt = jnp.take(weight1, gene_ix, axis=0).T.reshape(dim, genes, 1)
    bg = jnp.take(bias1, gene_ix, axis=0)
    out_t = _expression_tc(xt, wt, bg)
    return out_t.T
